# baseline (speedup 1.0000x reference)
import functools

import jax
import jax.numpy as jnp
from jax import lax
from jax.experimental import pallas as pl
from jax.experimental.pallas import tpu as pltpu
from jax.experimental.pallas import tpu_sc as plsc

T = 2048
H = 1024
I = 1024
E = 8
P = 2 * T
B = 128
NB = P // B
NT = NB + E

TB = 256

NC = 2
NS = 16
NW = NC * NS
ROWS_PER_W = P // NW
SC_CHUNK = 32


def _silu(v):
    return v * jax.nn.sigmoid(v)


def _router_kernel(l_ref, w1_ref, w2_ref, pos_ref, meta_ref):
    l = l_ref[...]
    col = lax.broadcasted_iota(jnp.int32, (T, E), 1)
    m1 = jnp.max(l, axis=1, keepdims=True)
    i1 = jnp.min(jnp.where(l == m1, col, E), axis=1, keepdims=True)
    l2 = jnp.where(col == i1, -jnp.inf, l)
    m2 = jnp.max(l2, axis=1, keepdims=True)
    i2 = jnp.min(jnp.where(l2 == m2, col, E), axis=1, keepdims=True)
    denom = jnp.maximum(m1 + m2, 1e-9)
    w1_ref[...] = m1 / denom
    w2_ref[...] = m2 / denom

    a = (jnp.where(col == i1, 1.0, 0.0)
         + jnp.where(col == i2, 1.0, 0.0))
    tr = lax.broadcasted_iota(jnp.int32, (T, T), 0)
    tc_ = lax.broadcasted_iota(jnp.int32, (T, T), 1)
    tri = jnp.where(tc_ <= tr, 1.0, 0.0)
    cinc = jnp.dot(tri, a, preferred_element_type=jnp.float32)
    cexc = cinc - a
    counts = cinc[T - 1:T, :]
    r8 = lax.broadcasted_iota(jnp.int32, (E, E), 0)
    c8 = lax.broadcasted_iota(jnp.int32, (E, E), 1)
    tri8 = jnp.where(r8 <= c8, 1.0, 0.0)
    offs = jnp.dot(counts, tri8, preferred_element_type=jnp.float32,
                   precision=lax.Precision.HIGHEST) - counts
    slot = cexc + offs
    pos0 = jnp.sum(jnp.where(col == i1, slot, 0.0), axis=1, keepdims=True)
    pos1 = jnp.sum(jnp.where(col == i2, slot, 0.0), axis=1, keepdims=True)
    pos_ref[0:T, :] = pos0.astype(jnp.int32)
    pos_ref[T:P, :] = pos1.astype(jnp.int32)

    excl = offs
    incl = offs + counts
    eids = lax.broadcasted_iota(jnp.int32, (1, E), 1).astype(jnp.float32)
    elast = jnp.max(jnp.where(counts > 0, eids, 0.0))

    ai = lax.broadcasted_iota(jnp.int32, (NB, 1), 0).astype(jnp.float32)
    astart = ai * B
    aown = jnp.sum(jnp.where(excl <= astart, 1.0, 0.0), axis=1,
                   keepdims=True) - 1.0
    erow = lax.broadcasted_iota(jnp.int32, (NB, E), 1).astype(jnp.float32)
    incl_own = jnp.sum(jnp.where(erow == aown, incl, 0.0), axis=1,
                       keepdims=True)
    ahi = jnp.minimum(incl_own, astart + B) - astart

    bs = excl
    bblk = jnp.floor(bs * (1.0 / B))
    bmod = bs - bblk * B
    bend = jnp.minimum(incl, (bblk + 1.0) * B)
    bvalid = (eids >= 1.0) & (bmod != 0.0) & (bend > bs)
    bvf = jnp.where(bvalid, 1.0, 0.0)
    bkey = jnp.where(bvalid, bs, 1e9)
    tr8s = jnp.where(r8 < c8, 1.0, 0.0)
    vcum = jnp.dot(bvf, tr8s, preferred_element_type=jnp.float32,
                   precision=lax.Precision.HIGHEST)
    nvalid = jnp.sum(bvf)

    rank_a = ai + jnp.sum(jnp.where(bkey < astart, 1.0, 0.0), axis=1,
                          keepdims=True)
    rank_b = jnp.where(bvalid, bblk + 1.0 + vcum,
                       jnp.where(eids == 0.0, float(NT - 1),
                                 NB + nvalid + eids - 1.0 - vcum))

    va = jnp.concatenate([ai, aown, jnp.zeros((NB, 1), jnp.float32), ahi,
                          jnp.ones((NB, 1), jnp.float32)], axis=1)
    vb_t = jnp.concatenate([jnp.where(bvalid, bblk, float(NB - 1)),
                            jnp.where(bvalid, eids, elast),
                            jnp.where(bvalid, bmod, 0.0),
                            jnp.where(bvalid, bend - bblk * B, 0.0),
                            jnp.zeros((1, E), jnp.float32)], axis=0)

    rows = lax.broadcasted_iota(jnp.int32, (NT, NB), 0).astype(jnp.float32)
    oa = jnp.where(rows == rank_a.reshape(1, NB), 1.0, 0.0)
    rows_b = lax.broadcasted_iota(jnp.int32, (NT, E), 0).astype(jnp.float32)
    ob = jnp.where(rows_b == rank_b, 1.0, 0.0)
    metaf = (jnp.dot(oa, va, preferred_element_type=jnp.float32,
                     precision=lax.Precision.HIGHEST)
             + jnp.dot(ob, vb_t.T, preferred_element_type=jnp.float32,
                       precision=lax.Precision.HIGHEST))
    ridx = lax.broadcasted_iota(jnp.int32, (NT, 5), 0).astype(jnp.float32)
    lane = lax.broadcasted_iota(jnp.int32, (NT, 5), 1).astype(jnp.float32)
    padvals = jnp.where(lane == 0.0, float(NB - 1),
                        jnp.where(lane == 1.0, elast, 0.0))
    metaf = jnp.where(ridx >= NB + nvalid, padvals, metaf)
    meta_ref[...] = metaf.astype(jnp.int32)


def _router(l):
    return pl.pallas_call(
        _router_kernel,
        out_shape=(
            jax.ShapeDtypeStruct((T, 1), jnp.float32),
            jax.ShapeDtypeStruct((T, 1), jnp.float32),
            jax.ShapeDtypeStruct((P, 1), jnp.int32),
            jax.ShapeDtypeStruct((NT, 5), jnp.int32),
        ),
    )(l)


def _sc_mesh():
    return plsc.VectorSubcoreMesh(core_axis_name="c", subcore_axis_name="s")


def _sc_scatter_rows(x, pos):
    @functools.partial(
        pl.kernel,
        out_type=jax.ShapeDtypeStruct((P, H), jnp.float32),
        mesh=_sc_mesh(),
        scratch_types=[pltpu.VMEM((SC_CHUNK,), jnp.int32),
                       pltpu.VMEM((SC_CHUNK, H), jnp.float32)],
    )
    def k(x_hbm, idx_hbm, out_hbm, idx_v, rows_v):
        wid = lax.axis_index("s") * NC + lax.axis_index("c")
        base = wid * ROWS_PER_W
        for cch in range(ROWS_PER_W // SC_CHUNK):
            off = base + cch * SC_CHUNK
            src = lax.rem(off, T)
            pltpu.sync_copy(idx_hbm.at[pl.ds(off, SC_CHUNK)], idx_v)
            pltpu.sync_copy(x_hbm.at[pl.ds(src, SC_CHUNK)], rows_v)
            pltpu.sync_copy(rows_v, out_hbm.at[idx_v])

    return k(x, pos)


def _sc_gather_rows(y, pos):
    @functools.partial(
        pl.kernel,
        out_type=jax.ShapeDtypeStruct((P, H), jnp.float32),
        mesh=_sc_mesh(),
        scratch_types=[pltpu.VMEM((SC_CHUNK,), jnp.int32),
                       pltpu.VMEM((SC_CHUNK, H), jnp.float32),
                       pltpu.SemaphoreType.DMA],
    )
    def k(y_hbm, idx_hbm, out_hbm, idx_v, rows_v, sem):
        wid = lax.axis_index("s") * NC + lax.axis_index("c")
        base = wid * ROWS_PER_W
        for cch in range(ROWS_PER_W // SC_CHUNK):
            off = base + cch * SC_CHUNK
            pltpu.sync_copy(idx_hbm.at[pl.ds(off, SC_CHUNK)], idx_v)
            pltpu.async_copy(y_hbm.at[idx_v], rows_v, sem).wait()
            pltpu.sync_copy(rows_v, out_hbm.at[pl.ds(off, SC_CHUNK)])

    return k(y, pos)


def _gemm_kernel(meta_ref, xs_ref, wi_ref, wo_ref, y_ref):
    i = pl.program_id(0)
    lo = meta_ref[i, 2]
    hi = meta_ref[i, 3]
    first = meta_ref[i, 4]

    @pl.when(hi > lo)
    def _():
        xs = xs_ref[...]
        wi = jnp.dot(xs, wi_ref[0], preferred_element_type=jnp.float32)
        x_proj = wi[:, :I]
        gate_e = wi[:, I:]
        y = jnp.dot(_silu(gate_e) * x_proj, wo_ref[0],
                    preferred_element_type=jnp.float32)
        rows = lax.broadcasted_iota(jnp.int32, (B, 1), 0)
        m = (rows >= lo) & (rows < hi)

        @pl.when(first == 1)
        def _():
            y_ref[...] = jnp.where(m, y, 0.0)

        @pl.when(first == 0)
        def _():
            y_ref[...] = jnp.where(m, y, y_ref[...])


def _grouped_gemm(meta, xs, Wi, Wo):
    grid_spec = pltpu.PrefetchScalarGridSpec(
        num_scalar_prefetch=1,
        grid=(NT,),
        in_specs=[
            pl.BlockSpec((B, H), lambda i, m: (m[i, 0], 0)),
            pl.BlockSpec((1, H, 2 * I), lambda i, m: (m[i, 1], 0, 0)),
            pl.BlockSpec((1, I, H), lambda i, m: (m[i, 1], 0, 0)),
        ],
        out_specs=pl.BlockSpec((B, H), lambda i, m: (m[i, 0], 0)),
    )
    return pl.pallas_call(
        _gemm_kernel,
        grid_spec=grid_spec,
        out_shape=jax.ShapeDtypeStruct((P, H), jnp.float32),
    )(meta, xs, Wi, Wo)


def _shared_kernel(x_ref, wis_ref, wos_ref, out_ref):
    h = jnp.dot(x_ref[...], wis_ref[...], preferred_element_type=jnp.float32)
    inp_s = h[:, :I]
    gate_s = h[:, I:]
    out_ref[...] = jnp.dot(_silu(inp_s) * gate_s, wos_ref[...],
                           preferred_element_type=jnp.float32)


def _shared(x, Wi_s, Wo_s):
    return pl.pallas_call(
        _shared_kernel,
        grid=(T // TB,),
        in_specs=[
            pl.BlockSpec((TB, H), lambda t: (t, 0)),
            pl.BlockSpec((H, 2 * I), lambda t: (0, 0)),
            pl.BlockSpec((I, H), lambda t: (0, 0)),
        ],
        out_specs=pl.BlockSpec((TB, H), lambda t: (t, 0)),
        out_shape=jax.ShapeDtypeStruct((T, H), jnp.float32),
    )(x, Wi_s, Wo_s)


def _combine_kernel(sh_ref, y0_ref, y1_ref, w1_ref, w2_ref, out_ref):
    out_ref[...] = (sh_ref[...] + w1_ref[...] * y0_ref[...]
                    + w2_ref[...] * y1_ref[...])


def _combine(shared, ys, w1, w2):
    nb0 = T // TB
    return pl.pallas_call(
        _combine_kernel,
        grid=(nb0,),
        in_specs=[
            pl.BlockSpec((TB, H), lambda t: (t, 0)),
            pl.BlockSpec((TB, H), lambda t: (t, 0)),
            pl.BlockSpec((TB, H), lambda t: (t + nb0, 0)),
            pl.BlockSpec((TB, 1), lambda t: (t, 0)),
            pl.BlockSpec((TB, 1), lambda t: (t, 0)),
        ],
        out_specs=pl.BlockSpec((TB, H), lambda t: (t, 0)),
        out_shape=jax.ShapeDtypeStruct((T, H), jnp.float32),
    )(shared, ys, ys, w1, w2)


@jax.jit
def kernel(x, gate_w, Wi, Wo, Wi_s, Wo_s):
    logits = jax.nn.sigmoid((x @ gate_w.T).astype(jnp.float32))
    w1, w2, pos, meta = _router(logits)
    pos1d = pos.reshape(P)
    xs = _sc_scatter_rows(x, pos1d)
    y = _grouped_gemm(meta, xs, Wi, Wo)
    ys = _sc_gather_rows(y, pos1d)
    shared = _shared(x, Wi_s, Wo_s)
    return _combine(shared, ys, w1, w2)


# speedup vs baseline: 1.0572x; 1.0572x over previous
import functools

import jax
import jax.numpy as jnp
from jax import lax
from jax.experimental import pallas as pl
from jax.experimental.pallas import tpu as pltpu
from jax.experimental.pallas import tpu_sc as plsc

T = 2048
H = 1024
I = 1024
E = 8
P = 2 * T
B = 256
NB = P // B
NT = NB + E

TB = 256

NC = 2
NS = 16
NW = NC * NS
ROWS_PER_W = P // NW
SC_CHUNK = 32


def _silu(v):
    return v * jax.nn.sigmoid(v)


def _router_kernel(l_ref, w1_ref, w2_ref, pos_ref, meta_ref):
    l = l_ref[...]
    col = lax.broadcasted_iota(jnp.int32, (T, E), 1)
    m1 = jnp.max(l, axis=1, keepdims=True)
    i1 = jnp.min(jnp.where(l == m1, col, E), axis=1, keepdims=True)
    l2 = jnp.where(col == i1, -jnp.inf, l)
    m2 = jnp.max(l2, axis=1, keepdims=True)
    i2 = jnp.min(jnp.where(l2 == m2, col, E), axis=1, keepdims=True)
    denom = jnp.maximum(m1 + m2, 1e-9)
    w1_ref[...] = m1 / denom
    w2_ref[...] = m2 / denom

    a = (jnp.where(col == i1, 1.0, 0.0)
         + jnp.where(col == i2, 1.0, 0.0))
    tr = lax.broadcasted_iota(jnp.int32, (T, T), 0)
    tc_ = lax.broadcasted_iota(jnp.int32, (T, T), 1)
    tri = jnp.where(tc_ <= tr, 1.0, 0.0)
    cinc = jnp.dot(tri, a, preferred_element_type=jnp.float32)
    cexc = cinc - a
    counts = cinc[T - 1:T, :]
    r8 = lax.broadcasted_iota(jnp.int32, (E, E), 0)
    c8 = lax.broadcasted_iota(jnp.int32, (E, E), 1)
    tri8 = jnp.where(r8 <= c8, 1.0, 0.0)
    offs = jnp.dot(counts, tri8, preferred_element_type=jnp.float32,
                   precision=lax.Precision.HIGHEST) - counts
    slot = cexc + offs
    pos0 = jnp.sum(jnp.where(col == i1, slot, 0.0), axis=1, keepdims=True)
    pos1 = jnp.sum(jnp.where(col == i2, slot, 0.0), axis=1, keepdims=True)
    pos_ref[0:T, :] = pos0.astype(jnp.int32)
    pos_ref[T:P, :] = pos1.astype(jnp.int32)

    excl = offs
    incl = offs + counts
    eids = lax.broadcasted_iota(jnp.int32, (1, E), 1).astype(jnp.float32)
    elast = jnp.max(jnp.where(counts > 0, eids, 0.0))

    ai = lax.broadcasted_iota(jnp.int32, (NB, 1), 0).astype(jnp.float32)
    astart = ai * B
    aown = jnp.sum(jnp.where(excl <= astart, 1.0, 0.0), axis=1,
                   keepdims=True) - 1.0
    erow = lax.broadcasted_iota(jnp.int32, (NB, E), 1).astype(jnp.float32)
    incl_own = jnp.sum(jnp.where(erow == aown, incl, 0.0), axis=1,
                       keepdims=True)
    ahi = jnp.minimum(incl_own, astart + B) - astart

    bs = excl
    bblk = jnp.floor(bs * (1.0 / B))
    bmod = bs - bblk * B
    bend = jnp.minimum(incl, (bblk + 1.0) * B)
    bvalid = (eids >= 1.0) & (bmod != 0.0) & (bend > bs)
    bvf = jnp.where(bvalid, 1.0, 0.0)
    bkey = jnp.where(bvalid, bs, 1e9)
    tr8s = jnp.where(r8 < c8, 1.0, 0.0)
    vcum = jnp.dot(bvf, tr8s, preferred_element_type=jnp.float32,
                   precision=lax.Precision.HIGHEST)
    nvalid = jnp.sum(bvf)

    rank_a = ai + jnp.sum(jnp.where(bkey < astart, 1.0, 0.0), axis=1,
                          keepdims=True)
    rank_b = jnp.where(bvalid, bblk + 1.0 + vcum,
                       jnp.where(eids == 0.0, float(NT - 1),
                                 NB + nvalid + eids - 1.0 - vcum))

    va = jnp.concatenate([ai, aown, jnp.zeros((NB, 1), jnp.float32), ahi,
                          jnp.ones((NB, 1), jnp.float32)], axis=1)
    vb_t = jnp.concatenate([jnp.where(bvalid, bblk, float(NB - 1)),
                            jnp.where(bvalid, eids, elast),
                            jnp.where(bvalid, bmod, 0.0),
                            jnp.where(bvalid, bend - bblk * B, 0.0),
                            jnp.zeros((1, E), jnp.float32)], axis=0)

    rows = lax.broadcasted_iota(jnp.int32, (NT, NB), 0).astype(jnp.float32)
    oa = jnp.where(rows == rank_a.reshape(1, NB), 1.0, 0.0)
    rows_b = lax.broadcasted_iota(jnp.int32, (NT, E), 0).astype(jnp.float32)
    ob = jnp.where(rows_b == rank_b, 1.0, 0.0)
    metaf = (jnp.dot(oa, va, preferred_element_type=jnp.float32,
                     precision=lax.Precision.HIGHEST)
             + jnp.dot(ob, vb_t.T, preferred_element_type=jnp.float32,
                       precision=lax.Precision.HIGHEST))
    ridx = lax.broadcasted_iota(jnp.int32, (NT, 5), 0).astype(jnp.float32)
    lane = lax.broadcasted_iota(jnp.int32, (NT, 5), 1).astype(jnp.float32)
    padvals = jnp.where(lane == 0.0, float(NB - 1),
                        jnp.where(lane == 1.0, elast, 0.0))
    metaf = jnp.where(ridx >= NB + nvalid, padvals, metaf)
    meta_ref[...] = metaf.astype(jnp.int32)


def _router(l):
    return pl.pallas_call(
        _router_kernel,
        out_shape=(
            jax.ShapeDtypeStruct((T, 1), jnp.float32),
            jax.ShapeDtypeStruct((T, 1), jnp.float32),
            jax.ShapeDtypeStruct((P, 1), jnp.int32),
            jax.ShapeDtypeStruct((NT, 5), jnp.int32),
        ),
    )(l)


def _sc_mesh():
    return plsc.VectorSubcoreMesh(core_axis_name="c", subcore_axis_name="s")


def _sc_scatter_rows(x, pos):
    @functools.partial(
        pl.kernel,
        out_type=jax.ShapeDtypeStruct((P, H), jnp.float32),
        mesh=_sc_mesh(),
        scratch_types=[pltpu.VMEM((SC_CHUNK,), jnp.int32),
                       pltpu.VMEM((SC_CHUNK, H), jnp.float32)],
    )
    def k(x_hbm, idx_hbm, out_hbm, idx_v, rows_v):
        wid = lax.axis_index("s") * NC + lax.axis_index("c")
        base = wid * ROWS_PER_W
        for cch in range(ROWS_PER_W // SC_CHUNK):
            off = base + cch * SC_CHUNK
            src = lax.rem(off, T)
            pltpu.sync_copy(idx_hbm.at[pl.ds(off, SC_CHUNK)], idx_v)
            pltpu.sync_copy(x_hbm.at[pl.ds(src, SC_CHUNK)], rows_v)
            pltpu.sync_copy(rows_v, out_hbm.at[idx_v])

    return k(x, pos)


def _sc_gather_rows(y, pos):
    @functools.partial(
        pl.kernel,
        out_type=jax.ShapeDtypeStruct((P, H), jnp.float32),
        mesh=_sc_mesh(),
        scratch_types=[pltpu.VMEM((SC_CHUNK,), jnp.int32),
                       pltpu.VMEM((SC_CHUNK, H), jnp.float32),
                       pltpu.SemaphoreType.DMA],
    )
    def k(y_hbm, idx_hbm, out_hbm, idx_v, rows_v, sem):
        wid = lax.axis_index("s") * NC + lax.axis_index("c")
        base = wid * ROWS_PER_W
        for cch in range(ROWS_PER_W // SC_CHUNK):
            off = base + cch * SC_CHUNK
            pltpu.sync_copy(idx_hbm.at[pl.ds(off, SC_CHUNK)], idx_v)
            pltpu.async_copy(y_hbm.at[idx_v], rows_v, sem).wait()
            pltpu.sync_copy(rows_v, out_hbm.at[pl.ds(off, SC_CHUNK)])

    return k(y, pos)


def _gemm_kernel(meta_ref, xs_ref, wi_ref, wo_ref, y_ref):
    i = pl.program_id(0)
    lo = meta_ref[i, 2]
    hi = meta_ref[i, 3]
    first = meta_ref[i, 4]

    @pl.when(hi > lo)
    def _():
        xs = xs_ref[...]
        wi = jnp.dot(xs, wi_ref[0], preferred_element_type=jnp.float32)
        x_proj = wi[:, :I]
        gate_e = wi[:, I:]
        y = jnp.dot(_silu(gate_e) * x_proj, wo_ref[0],
                    preferred_element_type=jnp.float32)
        rows = lax.broadcasted_iota(jnp.int32, (B, 1), 0)
        m = (rows >= lo) & (rows < hi)

        @pl.when(first == 1)
        def _():
            y_ref[...] = jnp.where(m, y, 0.0)

        @pl.when(first == 0)
        def _():
            y_ref[...] = jnp.where(m, y, y_ref[...])


def _grouped_gemm(meta, xs, Wi, Wo):
    grid_spec = pltpu.PrefetchScalarGridSpec(
        num_scalar_prefetch=1,
        grid=(NT,),
        in_specs=[
            pl.BlockSpec((B, H), lambda i, m: (m[i, 0], 0)),
            pl.BlockSpec((1, H, 2 * I), lambda i, m: (m[i, 1], 0, 0)),
            pl.BlockSpec((1, I, H), lambda i, m: (m[i, 1], 0, 0)),
        ],
        out_specs=pl.BlockSpec((B, H), lambda i, m: (m[i, 0], 0)),
    )
    return pl.pallas_call(
        _gemm_kernel,
        grid_spec=grid_spec,
        out_shape=jax.ShapeDtypeStruct((P, H), jnp.float32),
    )(meta, xs, Wi, Wo)


def _shared_kernel(x_ref, wis_ref, wos_ref, out_ref):
    h = jnp.dot(x_ref[...], wis_ref[...], preferred_element_type=jnp.float32)
    inp_s = h[:, :I]
    gate_s = h[:, I:]
    out_ref[...] = jnp.dot(_silu(inp_s) * gate_s, wos_ref[...],
                           preferred_element_type=jnp.float32)


def _shared(x, Wi_s, Wo_s):
    return pl.pallas_call(
        _shared_kernel,
        grid=(T // TB,),
        in_specs=[
            pl.BlockSpec((TB, H), lambda t: (t, 0)),
            pl.BlockSpec((H, 2 * I), lambda t: (0, 0)),
            pl.BlockSpec((I, H), lambda t: (0, 0)),
        ],
        out_specs=pl.BlockSpec((TB, H), lambda t: (t, 0)),
        out_shape=jax.ShapeDtypeStruct((T, H), jnp.float32),
    )(x, Wi_s, Wo_s)


def _combine_kernel(sh_ref, y0_ref, y1_ref, w1_ref, w2_ref, out_ref):
    out_ref[...] = (sh_ref[...] + w1_ref[...] * y0_ref[...]
                    + w2_ref[...] * y1_ref[...])


def _combine(shared, ys, w1, w2):
    nb0 = T // TB
    return pl.pallas_call(
        _combine_kernel,
        grid=(nb0,),
        in_specs=[
            pl.BlockSpec((TB, H), lambda t: (t, 0)),
            pl.BlockSpec((TB, H), lambda t: (t, 0)),
            pl.BlockSpec((TB, H), lambda t: (t + nb0, 0)),
            pl.BlockSpec((TB, 1), lambda t: (t, 0)),
            pl.BlockSpec((TB, 1), lambda t: (t, 0)),
        ],
        out_specs=pl.BlockSpec((TB, H), lambda t: (t, 0)),
        out_shape=jax.ShapeDtypeStruct((T, H), jnp.float32),
    )(shared, ys, ys, w1, w2)


@jax.jit
def kernel(x, gate_w, Wi, Wo, Wi_s, Wo_s):
    logits = jax.nn.sigmoid((x @ gate_w.T).astype(jnp.float32))
    w1, w2, pos, meta = _router(logits)
    pos1d = pos.reshape(P)
    xs = _sc_scatter_rows(x, pos1d)
    y = _grouped_gemm(meta, xs, Wi, Wo)
    ys = _sc_gather_rows(y, pos1d)
    shared = _shared(x, Wi_s, Wo_s)
    return _combine(shared, ys, w1, w2)


# speedup vs baseline: 1.0643x; 1.0067x over previous
import functools

import jax
import jax.numpy as jnp
from jax import lax
from jax.experimental import pallas as pl
from jax.experimental.pallas import tpu as pltpu
from jax.experimental.pallas import tpu_sc as plsc

T = 2048
H = 1024
I = 1024
E = 8
P = 2 * T
B = 256
NB = P // B
NT = NB + E
PA = NT * B

TB = 256

NC = 2
NS = 16
NW = NC * NS
ROWS_PER_W = P // NW
SC_CHUNK = 32


def _silu(v):
    return v * jax.nn.sigmoid(v)


def _router_kernel(l_ref, w1_ref, w2_ref, pos_ref, meta_ref):
    l = l_ref[...]
    col = lax.broadcasted_iota(jnp.int32, (T, E), 1)
    m1 = jnp.max(l, axis=1, keepdims=True)
    i1 = jnp.min(jnp.where(l == m1, col, E), axis=1, keepdims=True)
    l2 = jnp.where(col == i1, -jnp.inf, l)
    m2 = jnp.max(l2, axis=1, keepdims=True)
    i2 = jnp.min(jnp.where(l2 == m2, col, E), axis=1, keepdims=True)
    denom = jnp.maximum(m1 + m2, 1e-9)
    w1_ref[...] = m1 / denom
    w2_ref[...] = m2 / denom

    a = (jnp.where(col == i1, 1.0, 0.0)
         + jnp.where(col == i2, 1.0, 0.0))
    tr = lax.broadcasted_iota(jnp.int32, (T, T), 0)
    tc_ = lax.broadcasted_iota(jnp.int32, (T, T), 1)
    tri = jnp.where(tc_ <= tr, 1.0, 0.0)
    cinc = jnp.dot(tri, a, preferred_element_type=jnp.float32)
    cexc = cinc - a
    counts = cinc[T - 1:T, :]
    r8 = lax.broadcasted_iota(jnp.int32, (E, E), 0)
    c8 = lax.broadcasted_iota(jnp.int32, (E, E), 1)
    tr8s = jnp.where(r8 < c8, 1.0, 0.0)
    blocks = jnp.ceil(counts * (1.0 / B))
    tb = jnp.dot(blocks, tr8s, preferred_element_type=jnp.float32,
                 precision=lax.Precision.HIGHEST)
    offs = tb * B
    slot = cexc + offs
    pos0 = jnp.sum(jnp.where(col == i1, slot, 0.0), axis=1, keepdims=True)
    pos1 = jnp.sum(jnp.where(col == i2, slot, 0.0), axis=1, keepdims=True)
    pos_ref[0:T, :] = pos0.astype(jnp.int32)
    pos_ref[T:P, :] = pos1.astype(jnp.int32)

    rr = lax.broadcasted_iota(jnp.int32, (NT, 1), 0).astype(jnp.float32)
    own = jnp.sum(jnp.where(tb <= rr, 1.0, 0.0), axis=1,
                  keepdims=True) - 1.0
    erow = lax.broadcasted_iota(jnp.int32, (NT, E), 1).astype(jnp.float32)
    c_own = jnp.sum(jnp.where(erow == own, counts, 0.0), axis=1,
                    keepdims=True)
    tb_own = jnp.sum(jnp.where(erow == own, tb, 0.0), axis=1,
                     keepdims=True)
    hi = jnp.clip(c_own - (rr - tb_own) * B, 0.0, float(B))
    metaf = jnp.concatenate([rr, own, jnp.zeros((NT, 1), jnp.float32), hi,
                             jnp.ones((NT, 1), jnp.float32)], axis=1)
    meta_ref[...] = metaf.astype(jnp.int32)


def _router(l):
    return pl.pallas_call(
        _router_kernel,
        out_shape=(
            jax.ShapeDtypeStruct((T, 1), jnp.float32),
            jax.ShapeDtypeStruct((T, 1), jnp.float32),
            jax.ShapeDtypeStruct((P, 1), jnp.int32),
            jax.ShapeDtypeStruct((NT, 5), jnp.int32),
        ),
    )(l)


def _sc_mesh():
    return plsc.VectorSubcoreMesh(core_axis_name="c", subcore_axis_name="s")


def _sc_scatter_rows(x, pos):
    @functools.partial(
        pl.kernel,
        out_type=jax.ShapeDtypeStruct((PA, H), jnp.float32),
        mesh=_sc_mesh(),
        scratch_types=[pltpu.VMEM((SC_CHUNK,), jnp.int32),
                       pltpu.VMEM((SC_CHUNK, H), jnp.float32)],
    )
    def k(x_hbm, idx_hbm, out_hbm, idx_v, rows_v):
        wid = lax.axis_index("s") * NC + lax.axis_index("c")
        base = wid * ROWS_PER_W
        for cch in range(ROWS_PER_W // SC_CHUNK):
            off = base + cch * SC_CHUNK
            src = lax.rem(off, T)
            pltpu.sync_copy(idx_hbm.at[pl.ds(off, SC_CHUNK)], idx_v)
            pltpu.sync_copy(x_hbm.at[pl.ds(src, SC_CHUNK)], rows_v)
            pltpu.sync_copy(rows_v, out_hbm.at[idx_v])

    return k(x, pos)


def _sc_gather_rows(y, pos):
    @functools.partial(
        pl.kernel,
        out_type=jax.ShapeDtypeStruct((P, H), jnp.float32),
        mesh=_sc_mesh(),
        scratch_types=[pltpu.VMEM((SC_CHUNK,), jnp.int32),
                       pltpu.VMEM((SC_CHUNK, H), jnp.float32),
                       pltpu.SemaphoreType.DMA],
    )
    def k(y_hbm, idx_hbm, out_hbm, idx_v, rows_v, sem):
        wid = lax.axis_index("s") * NC + lax.axis_index("c")
        base = wid * ROWS_PER_W
        for cch in range(ROWS_PER_W // SC_CHUNK):
            off = base + cch * SC_CHUNK
            pltpu.sync_copy(idx_hbm.at[pl.ds(off, SC_CHUNK)], idx_v)
            pltpu.async_copy(y_hbm.at[idx_v], rows_v, sem).wait()
            pltpu.sync_copy(rows_v, out_hbm.at[pl.ds(off, SC_CHUNK)])

    return k(y, pos)


def _gemm_kernel(meta_ref, xs_ref, wi_ref, wo_ref, y_ref):
    i = pl.program_id(0)
    hi = meta_ref[i, 3]

    @pl.when(hi > 0)
    def _():
        xs = xs_ref[...]
        wi = jnp.dot(xs, wi_ref[0], preferred_element_type=jnp.float32)
        x_proj = wi[:, :I]
        gate_e = wi[:, I:]
        y = jnp.dot(_silu(gate_e) * x_proj, wo_ref[0],
                    preferred_element_type=jnp.float32)
        rows = lax.broadcasted_iota(jnp.int32, (B, 1), 0)
        y_ref[...] = jnp.where(rows < hi, y, 0.0)


def _grouped_gemm(meta, xs, Wi, Wo):
    grid_spec = pltpu.PrefetchScalarGridSpec(
        num_scalar_prefetch=1,
        grid=(NT,),
        in_specs=[
            pl.BlockSpec((B, H), lambda i, m: (m[i, 0], 0)),
            pl.BlockSpec((1, H, 2 * I), lambda i, m: (m[i, 1], 0, 0)),
            pl.BlockSpec((1, I, H), lambda i, m: (m[i, 1], 0, 0)),
        ],
        out_specs=pl.BlockSpec((B, H), lambda i, m: (m[i, 0], 0)),
    )
    return pl.pallas_call(
        _gemm_kernel,
        grid_spec=grid_spec,
        out_shape=jax.ShapeDtypeStruct((PA, H), jnp.float32),
    )(meta, xs, Wi, Wo)


def _shared_kernel(x_ref, wis_ref, wos_ref, out_ref):
    h = jnp.dot(x_ref[...], wis_ref[...], preferred_element_type=jnp.float32)
    inp_s = h[:, :I]
    gate_s = h[:, I:]
    out_ref[...] = jnp.dot(_silu(inp_s) * gate_s, wos_ref[...],
                           preferred_element_type=jnp.float32)


def _shared(x, Wi_s, Wo_s):
    return pl.pallas_call(
        _shared_kernel,
        grid=(T // TB,),
        in_specs=[
            pl.BlockSpec((TB, H), lambda t: (t, 0)),
            pl.BlockSpec((H, 2 * I), lambda t: (0, 0)),
            pl.BlockSpec((I, H), lambda t: (0, 0)),
        ],
        out_specs=pl.BlockSpec((TB, H), lambda t: (t, 0)),
        out_shape=jax.ShapeDtypeStruct((T, H), jnp.float32),
    )(x, Wi_s, Wo_s)


def _combine_kernel(sh_ref, y0_ref, y1_ref, w1_ref, w2_ref, out_ref):
    out_ref[...] = (sh_ref[...] + w1_ref[...] * y0_ref[...]
                    + w2_ref[...] * y1_ref[...])


def _combine(shared, ys, w1, w2):
    nb0 = T // TB
    return pl.pallas_call(
        _combine_kernel,
        grid=(nb0,),
        in_specs=[
            pl.BlockSpec((TB, H), lambda t: (t, 0)),
            pl.BlockSpec((TB, H), lambda t: (t, 0)),
            pl.BlockSpec((TB, H), lambda t: (t + nb0, 0)),
            pl.BlockSpec((TB, 1), lambda t: (t, 0)),
            pl.BlockSpec((TB, 1), lambda t: (t, 0)),
        ],
        out_specs=pl.BlockSpec((TB, H), lambda t: (t, 0)),
        out_shape=jax.ShapeDtypeStruct((T, H), jnp.float32),
    )(shared, ys, ys, w1, w2)


@jax.jit
def kernel(x, gate_w, Wi, Wo, Wi_s, Wo_s):
    logits = jax.nn.sigmoid((x @ gate_w.T).astype(jnp.float32))
    w1, w2, pos, meta = _router(logits)
    pos1d = pos.reshape(P)
    xs = _sc_scatter_rows(x, pos1d)
    y = _grouped_gemm(meta, xs, Wi, Wo)
    ys = _sc_gather_rows(y, pos1d)
    shared = _shared(x, Wi_s, Wo_s)
    return _combine(shared, ys, w1, w2)


# speedup vs baseline: 1.0816x; 1.0163x over previous
import functools

import jax
import jax.numpy as jnp
from jax import lax
from jax.experimental import pallas as pl
from jax.experimental.pallas import tpu as pltpu
from jax.experimental.pallas import tpu_sc as plsc

T = 2048
H = 1024
I = 1024
E = 8
P = 2 * T
B = 256
NB = P // B
NT = NB + E
PA = NT * B

TB = 512

NC = 2
NS = 16
NW = NC * NS
ROWS_PER_W = P // NW
SC_CHUNK = 32


def _silu(v):
    return v * jax.nn.sigmoid(v)


def _router_kernel(l_ref, w1_ref, w2_ref, pos_ref, meta_ref):
    l = l_ref[...]
    col = lax.broadcasted_iota(jnp.int32, (T, E), 1)
    m1 = jnp.max(l, axis=1, keepdims=True)
    i1 = jnp.min(jnp.where(l == m1, col, E), axis=1, keepdims=True)
    l2 = jnp.where(col == i1, -jnp.inf, l)
    m2 = jnp.max(l2, axis=1, keepdims=True)
    i2 = jnp.min(jnp.where(l2 == m2, col, E), axis=1, keepdims=True)
    denom = jnp.maximum(m1 + m2, 1e-9)
    w1_ref[...] = m1 / denom
    w2_ref[...] = m2 / denom

    a = (jnp.where(col == i1, 1.0, 0.0)
         + jnp.where(col == i2, 1.0, 0.0))
    tr = lax.broadcasted_iota(jnp.int32, (T, T), 0)
    tc_ = lax.broadcasted_iota(jnp.int32, (T, T), 1)
    tri = jnp.where(tc_ <= tr, 1.0, 0.0)
    cinc = jnp.dot(tri, a, preferred_element_type=jnp.float32)
    cexc = cinc - a
    counts = cinc[T - 1:T, :]
    r8 = lax.broadcasted_iota(jnp.int32, (E, E), 0)
    c8 = lax.broadcasted_iota(jnp.int32, (E, E), 1)
    tr8s = jnp.where(r8 < c8, 1.0, 0.0)
    blocks = jnp.ceil(counts * (1.0 / B))
    tb = jnp.dot(blocks, tr8s, preferred_element_type=jnp.float32,
                 precision=lax.Precision.HIGHEST)
    offs = tb * B
    slot = cexc + offs
    pos0 = jnp.sum(jnp.where(col == i1, slot, 0.0), axis=1, keepdims=True)
    pos1 = jnp.sum(jnp.where(col == i2, slot, 0.0), axis=1, keepdims=True)
    pos_ref[0:T, :] = pos0.astype(jnp.int32)
    pos_ref[T:P, :] = pos1.astype(jnp.int32)

    rr = lax.broadcasted_iota(jnp.int32, (NT, 1), 0).astype(jnp.float32)
    own = jnp.sum(jnp.where(tb <= rr, 1.0, 0.0), axis=1,
                  keepdims=True) - 1.0
    erow = lax.broadcasted_iota(jnp.int32, (NT, E), 1).astype(jnp.float32)
    c_own = jnp.sum(jnp.where(erow == own, counts, 0.0), axis=1,
                    keepdims=True)
    tb_own = jnp.sum(jnp.where(erow == own, tb, 0.0), axis=1,
                     keepdims=True)
    hi = jnp.clip(c_own - (rr - tb_own) * B, 0.0, float(B))
    metaf = jnp.concatenate([rr, own, jnp.zeros((NT, 1), jnp.float32), hi,
                             jnp.ones((NT, 1), jnp.float32)], axis=1)
    meta_ref[...] = metaf.astype(jnp.int32)


def _router(l):
    return pl.pallas_call(
        _router_kernel,
        out_shape=(
            jax.ShapeDtypeStruct((T, 1), jnp.float32),
            jax.ShapeDtypeStruct((T, 1), jnp.float32),
            jax.ShapeDtypeStruct((P, 1), jnp.int32),
            jax.ShapeDtypeStruct((NT, 5), jnp.int32),
        ),
    )(l)


def _sc_mesh():
    return plsc.VectorSubcoreMesh(core_axis_name="c", subcore_axis_name="s")


def _sc_scatter_rows(x, pos):
    @functools.partial(
        pl.kernel,
        out_type=jax.ShapeDtypeStruct((PA, H), jnp.float32),
        mesh=_sc_mesh(),
        scratch_types=[pltpu.VMEM((SC_CHUNK,), jnp.int32),
                       pltpu.VMEM((SC_CHUNK, H), jnp.float32)],
    )
    def k(x_hbm, idx_hbm, out_hbm, idx_v, rows_v):
        wid = lax.axis_index("s") * NC + lax.axis_index("c")
        base = wid * ROWS_PER_W
        for cch in range(ROWS_PER_W // SC_CHUNK):
            off = base + cch * SC_CHUNK
            src = lax.rem(off, T)
            pltpu.sync_copy(idx_hbm.at[pl.ds(off, SC_CHUNK)], idx_v)
            pltpu.sync_copy(x_hbm.at[pl.ds(src, SC_CHUNK)], rows_v)
            pltpu.sync_copy(rows_v, out_hbm.at[idx_v])

    return k(x, pos)


def _sc_gather_rows(y, pos):
    @functools.partial(
        pl.kernel,
        out_type=jax.ShapeDtypeStruct((P, H), jnp.float32),
        mesh=_sc_mesh(),
        scratch_types=[pltpu.VMEM((SC_CHUNK,), jnp.int32),
                       pltpu.VMEM((SC_CHUNK, H), jnp.float32),
                       pltpu.SemaphoreType.DMA],
    )
    def k(y_hbm, idx_hbm, out_hbm, idx_v, rows_v, sem):
        wid = lax.axis_index("s") * NC + lax.axis_index("c")
        base = wid * ROWS_PER_W
        for cch in range(ROWS_PER_W // SC_CHUNK):
            off = base + cch * SC_CHUNK
            pltpu.sync_copy(idx_hbm.at[pl.ds(off, SC_CHUNK)], idx_v)
            pltpu.async_copy(y_hbm.at[idx_v], rows_v, sem).wait()
            pltpu.sync_copy(rows_v, out_hbm.at[pl.ds(off, SC_CHUNK)])

    return k(y, pos)


def _gemm_kernel(meta_ref, xs_ref, wi_ref, wo_ref, y_ref):
    i = pl.program_id(0)
    hi = meta_ref[i, 3]

    @pl.when(hi > 0)
    def _():
        xs = xs_ref[...]
        wi = jnp.dot(xs, wi_ref[0], preferred_element_type=jnp.float32)
        x_proj = wi[:, :I]
        gate_e = wi[:, I:]
        y = jnp.dot(_silu(gate_e) * x_proj, wo_ref[0],
                    preferred_element_type=jnp.float32)
        rows = lax.broadcasted_iota(jnp.int32, (B, 1), 0)
        y_ref[...] = jnp.where(rows < hi, y, 0.0)


def _grouped_gemm(meta, xs, Wi, Wo):
    grid_spec = pltpu.PrefetchScalarGridSpec(
        num_scalar_prefetch=1,
        grid=(NT,),
        in_specs=[
            pl.BlockSpec((B, H), lambda i, m: (m[i, 0], 0)),
            pl.BlockSpec((1, H, 2 * I), lambda i, m: (m[i, 1], 0, 0)),
            pl.BlockSpec((1, I, H), lambda i, m: (m[i, 1], 0, 0)),
        ],
        out_specs=pl.BlockSpec((B, H), lambda i, m: (m[i, 0], 0)),
    )
    return pl.pallas_call(
        _gemm_kernel,
        grid_spec=grid_spec,
        out_shape=jax.ShapeDtypeStruct((PA, H), jnp.float32),
    )(meta, xs, Wi, Wo)


def _shared_kernel(x_ref, wis_ref, wos_ref, out_ref):
    h = jnp.dot(x_ref[...], wis_ref[...], preferred_element_type=jnp.float32)
    inp_s = h[:, :I]
    gate_s = h[:, I:]
    out_ref[...] = jnp.dot(_silu(inp_s) * gate_s, wos_ref[...],
                           preferred_element_type=jnp.float32)


def _shared(x, Wi_s, Wo_s):
    return pl.pallas_call(
        _shared_kernel,
        grid=(T // TB,),
        in_specs=[
            pl.BlockSpec((TB, H), lambda t: (t, 0)),
            pl.BlockSpec((H, 2 * I), lambda t: (0, 0)),
            pl.BlockSpec((I, H), lambda t: (0, 0)),
        ],
        out_specs=pl.BlockSpec((TB, H), lambda t: (t, 0)),
        out_shape=jax.ShapeDtypeStruct((T, H), jnp.float32),
    )(x, Wi_s, Wo_s)


def _combine_kernel(sh_ref, y0_ref, y1_ref, w1_ref, w2_ref, out_ref):
    out_ref[...] = (sh_ref[...] + w1_ref[...] * y0_ref[...]
                    + w2_ref[...] * y1_ref[...])


def _combine(shared, ys, w1, w2):
    nb0 = T // TB
    return pl.pallas_call(
        _combine_kernel,
        grid=(nb0,),
        in_specs=[
            pl.BlockSpec((TB, H), lambda t: (t, 0)),
            pl.BlockSpec((TB, H), lambda t: (t, 0)),
            pl.BlockSpec((TB, H), lambda t: (t + nb0, 0)),
            pl.BlockSpec((TB, 1), lambda t: (t, 0)),
            pl.BlockSpec((TB, 1), lambda t: (t, 0)),
        ],
        out_specs=pl.BlockSpec((TB, H), lambda t: (t, 0)),
        out_shape=jax.ShapeDtypeStruct((T, H), jnp.float32),
    )(shared, ys, ys, w1, w2)


@jax.jit
def kernel(x, gate_w, Wi, Wo, Wi_s, Wo_s):
    logits = jax.nn.sigmoid((x @ gate_w.T).astype(jnp.float32))
    w1, w2, pos, meta = _router(logits)
    pos1d = pos.reshape(P)
    xs = _sc_scatter_rows(x, pos1d)
    y = _grouped_gemm(meta, xs, Wi, Wo)
    ys = _sc_gather_rows(y, pos1d)
    shared = _shared(x, Wi_s, Wo_s)
    return _combine(shared, ys, w1, w2)
